# baseline (speedup 1.0000x reference)
import functools

import jax
import jax.numpy as jnp
from jax import lax
from jax.experimental import pallas as pl
from jax.experimental.pallas import tpu as pltpu
from jax.experimental.pallas import tpu_sc as plsc

N = 10000
E = 320000
D = 128
C = 16

L = 16
NC = 2
NS = 16
NW = NC * NS
CHUNK = 128
NCH = 2560
E2 = NCH * CHUNK
CPT = NCH // NW
NBUF = 8
NP = 10240
RPT = NP // NS


def _edge_pass(q, kv, edges):
    mesh = plsc.VectorSubcoreMesh(core_axis_name="c", subcore_axis_name="s")

    @functools.partial(
        pl.kernel,
        mesh=mesh,
        compiler_params=pltpu.CompilerParams(
            needs_layout_passes=False, use_tc_tiling_on_sc=False),
        out_type=jax.ShapeDtypeStruct((NC, NP, 2 * C), jnp.float32),
        scratch_types=[
            pltpu.VMEM((2, CPT, CHUNK), jnp.int32),
            pltpu.VMEM((NBUF, CHUNK, C), jnp.float32),
            pltpu.VMEM((NBUF, CHUNK, C), jnp.uint32),
            pltpu.VMEM((NBUF, CHUNK, 2 * C), jnp.float32),
            pltpu.VMEM((RPT, 2 * C), jnp.float32),
            pltpu.VMEM_SHARED((NP, 2 * C), jnp.float32),
        ] + [pltpu.SemaphoreType.DMA] * 17,
    )
    def kern(q_hbm, kv_hbm, e_hbm, out_hbm,
             idx, qb, kvb, ob, zbuf, acc, *sems):
        gsem = list(sems[:NBUF])
        ssem = list(sems[NBUF:2 * NBUF])
        isem = sems[2 * NBUF]
        cid = lax.axis_index("c")
        sid = lax.axis_index("s")
        wid = sid * NC + cid

        pltpu.async_copy(e_hbm.at[:, pl.ds(wid * CPT, CPT)], idx, isem)

        def zrow(r, carry):
            zbuf[r, 0:C] = jnp.zeros((L,), jnp.float32)
            zbuf[r, C:2 * C] = jnp.zeros((L,), jnp.float32)
            return carry

        lax.fori_loop(0, RPT, zrow, 0)
        pltpu.sync_copy(zbuf, acc.at[pl.ds(sid * RPT, RPT)])
        pltpu.make_async_copy(e_hbm.at[:, pl.ds(wid * CPT, CPT)], idx,
                              isem).wait()
        plsc.subcore_barrier()

        def fire_gathers(b, j):
            pltpu.async_copy(q_hbm.at[idx.at[1, j]], qb.at[b], gsem[b])
            pltpu.async_copy(kv_hbm.at[idx.at[0, j]], kvb.at[b], gsem[b])

        for b in range(NBUF):
            fire_gathers(b, b)

        def grp(g, carry):
            for b in range(NBUF):
                j = g * NBUF + b
                pltpu.make_async_copy(q_hbm.at[idx.at[1, j]], qb.at[b],
                                      gsem[b]).wait()
                pltpu.make_async_copy(kv_hbm.at[idx.at[0, j]], kvb.at[b],
                                      gsem[b]).wait()

                @pl.when(g > 0)
                def _():
                    pltpu.make_async_copy(ob.at[b], acc.at[idx.at[1, j]],
                                          ssem[b]).wait()

                for r in range(CHUNK):
                    row = plsc.bitcast(kvb[b, r, :], jnp.bfloat16)
                    kvec, vvec = plsc.unpack(
                        row, format=plsc.PackFormat.INTERLEAVED)
                    prod = qb[b, r, :] * kvec
                    s = jnp.sum(prod)
                    wvec = jnp.exp(jnp.full((L,), s, jnp.float32) * 0.25)
                    ob[b, r, 0:C] = vvec * wvec
                    ob[b, r, C:2 * C] = wvec

                pltpu.async_copy(ob.at[b], acc.at[idx.at[1, j]], ssem[b],
                                 add=True)

                @pl.when(g < CPT // NBUF - 1)
                def _():
                    fire_gathers(b, j + NBUF)

            return carry

        lax.fori_loop(0, CPT // NBUF, grp, 0)
        for b in range(NBUF):
            pltpu.make_async_copy(ob.at[b], acc.at[idx.at[1, CPT - NBUF + b]],
                                  ssem[b]).wait()
        plsc.subcore_barrier()
        pltpu.sync_copy(acc.at[pl.ds(sid * RPT, RPT)],
                        out_hbm.at[cid, pl.ds(sid * RPT, RPT)])

    return kern(q, kv, edges)


def _proj1(x, Wq, bq, Wk, bk, Wv, bv, Ws, bs):
    def body(x_r, wq, bqr, wk, bkr, wv, bvr, ws, bsr, q_o, kv_o, skip_o):
        xv = x_r[...]
        q_o[0:N, :] = jnp.dot(xv, wq[...], preferred_element_type=jnp.float32) + bqr[...]
        q_o[N:NP, :] = jnp.zeros((NP - N, C), jnp.float32)
        kk = jnp.dot(xv, wk[...], preferred_element_type=jnp.float32) + bkr[...]
        vv = jnp.dot(xv, wv[...], preferred_element_type=jnp.float32) + bvr[...]
        kb = jax.lax.bitcast_convert_type(kk.astype(jnp.bfloat16), jnp.uint16).astype(jnp.uint32)
        vb = jax.lax.bitcast_convert_type(vv.astype(jnp.bfloat16), jnp.uint16).astype(jnp.uint32)
        kv_o[0:N, :] = kb | (vb << 16)
        kv_o[N:NP, :] = jnp.zeros((NP - N, C), jnp.uint32)
        skip_o[...] = jnp.dot(xv, ws[...], preferred_element_type=jnp.float32) + bsr[...]

    return pl.pallas_call(
        body,
        out_shape=(
            jax.ShapeDtypeStruct((NP, C), jnp.float32),
            jax.ShapeDtypeStruct((NP, C), jnp.uint32),
            jax.ShapeDtypeStruct((N, C), jnp.float32),
        ),
    )(x, Wq, bq.reshape(1, C), Wk, bk.reshape(1, C),
      Wv, bv.reshape(1, C), Ws, bs.reshape(1, C))


def _combine_proj2(accs, skip, Wq, bq, Wk, bk, Wv, bv, Ws, bs):
    def body(a_r, s_r, wq, bqr, wk, bkr, wv, bvr, ws, bsr, q_o, kv_o, skip_o):
        numer = a_r[0, 0:N, 0:C] + a_r[1, 0:N, 0:C]
        den = a_r[0, 0:N, C:C + 1] + a_r[1, 0:N, C:C + 1]
        h = jax.nn.relu(numer / jnp.maximum(den, 1e-30) + s_r[...])
        q_o[0:N, :] = jnp.dot(h, wq[...], preferred_element_type=jnp.float32) + bqr[...]
        q_o[N:NP, :] = jnp.zeros((NP - N, C), jnp.float32)
        kk = jnp.dot(h, wk[...], preferred_element_type=jnp.float32) + bkr[...]
        vv = jnp.dot(h, wv[...], preferred_element_type=jnp.float32) + bvr[...]
        kb = jax.lax.bitcast_convert_type(kk.astype(jnp.bfloat16), jnp.uint16).astype(jnp.uint32)
        vb = jax.lax.bitcast_convert_type(vv.astype(jnp.bfloat16), jnp.uint16).astype(jnp.uint32)
        kv_o[0:N, :] = kb | (vb << 16)
        kv_o[N:NP, :] = jnp.zeros((NP - N, C), jnp.uint32)
        skip_o[...] = jnp.dot(h, ws[...], preferred_element_type=jnp.float32) + bsr[...]

    return pl.pallas_call(
        body,
        out_shape=(
            jax.ShapeDtypeStruct((NP, C), jnp.float32),
            jax.ShapeDtypeStruct((NP, C), jnp.uint32),
            jax.ShapeDtypeStruct((N, C), jnp.float32),
        ),
    )(accs, skip, Wq, bq.reshape(1, C), Wk, bk.reshape(1, C),
      Wv, bv.reshape(1, C), Ws, bs.reshape(1, C))


def _combine_mlp(accs, skip, W1, b1, W2, b2, W3, b3):
    def elu(t):
        return jnp.where(t > 0, t, jnp.exp(jnp.minimum(t, 0.0)) - 1.0)

    def body(a_r, s_r, w1, b1r, w2, b2r, w3, b3r, o_r):
        numer = a_r[0, 0:N, 0:C] + a_r[1, 0:N, 0:C]
        den = a_r[0, 0:N, C:C + 1] + a_r[1, 0:N, C:C + 1]
        h = jax.nn.relu(numer / jnp.maximum(den, 1e-30) + s_r[...])
        h = elu(jnp.dot(h, w1[...], preferred_element_type=jnp.float32) + b1r[...])
        h = elu(jnp.dot(h, w2[...], preferred_element_type=jnp.float32) + b2r[...])
        h = elu(jnp.dot(h, w3[...], preferred_element_type=jnp.float32) + b3r[...])
        m = jnp.max(h, axis=1, keepdims=True)
        o_r[...] = h - m - jnp.log(jnp.sum(jnp.exp(h - m), axis=1, keepdims=True))

    return pl.pallas_call(
        body,
        out_shape=jax.ShapeDtypeStruct((N, 10), jnp.float32),
    )(accs, skip, W1, b1.reshape(1, -1), W2, b2.reshape(1, -1),
      W3, b3.reshape(1, -1))


def kernel(x, edge_index, Wq1, bq1, Wk1, bk1, Wv1, bv1, Ws1, bs1,
           Wq2, bq2, Wk2, bk2, Wv2, bv2, Ws2, bs2,
           W1, b1, W2, b2, W3, b3):
    ei = edge_index.astype(jnp.int32)
    pad = jnp.full((2, E2 - E), N, jnp.int32)
    edges = jnp.concatenate([ei, pad], axis=1).reshape(2, NCH, CHUNK)
    q1, kv1, skip1 = _proj1(x.astype(jnp.float32), Wq1, bq1, Wk1, bk1,
                            Wv1, bv1, Ws1, bs1)
    acc1 = _edge_pass(q1, kv1, edges)
    q2, kv2, skip2 = _combine_proj2(acc1, skip1, Wq2, bq2, Wk2, bk2,
                                    Wv2, bv2, Ws2, bs2)
    acc2 = _edge_pass(q2, kv2, edges)
    return _combine_mlp(acc2, skip2, W1, b1, W2, b2, W3, b3)


# speedup vs baseline: 1.0785x; 1.0785x over previous
import functools

import jax
import jax.numpy as jnp
from jax import lax
from jax.experimental import pallas as pl
from jax.experimental.pallas import tpu as pltpu
from jax.experimental.pallas import tpu_sc as plsc

N = 10000
E = 320000
D = 128
C = 16

L = 16
NC = 2
NS = 16
NW = NC * NS
CHUNK = 128
NCH = 2560
E2 = NCH * CHUNK
CPT = NCH // NW
NBUF = 2
NP = 10240
RPT = NP // NS


def _edge_pass(q, kv, edges):
    mesh = plsc.VectorSubcoreMesh(core_axis_name="c", subcore_axis_name="s")

    @functools.partial(
        pl.kernel,
        mesh=mesh,
        compiler_params=pltpu.CompilerParams(
            needs_layout_passes=False, use_tc_tiling_on_sc=False),
        out_type=jax.ShapeDtypeStruct((NC, NP, 2 * C), jnp.float32),
        scratch_types=[
            pltpu.VMEM((2, CPT, CHUNK), jnp.int32),
            pltpu.VMEM((NBUF, CHUNK, C), jnp.float32),
            pltpu.VMEM((NBUF, CHUNK, C), jnp.uint32),
            pltpu.VMEM((NBUF, CHUNK, 2 * C), jnp.float32),
            pltpu.VMEM((RPT, 2 * C), jnp.float32),
            pltpu.VMEM_SHARED((NP, 2 * C), jnp.float32),
        ] + [pltpu.SemaphoreType.DMA] * 17,
    )
    def kern(q_hbm, kv_hbm, e_hbm, out_hbm,
             idx, qb, kvb, ob, zbuf, acc, *sems):
        gsem = list(sems[:NBUF])
        ssem = list(sems[NBUF:2 * NBUF])
        isem = sems[2 * NBUF]
        cid = lax.axis_index("c")
        sid = lax.axis_index("s")
        wid = sid * NC + cid

        pltpu.async_copy(e_hbm.at[:, pl.ds(wid * CPT, CPT)], idx, isem)

        def zrow(r, carry):
            zbuf[r, 0:C] = jnp.zeros((L,), jnp.float32)
            zbuf[r, C:2 * C] = jnp.zeros((L,), jnp.float32)
            return carry

        lax.fori_loop(0, RPT, zrow, 0)
        pltpu.sync_copy(zbuf, acc.at[pl.ds(sid * RPT, RPT)])
        pltpu.make_async_copy(e_hbm.at[:, pl.ds(wid * CPT, CPT)], idx,
                              isem).wait()
        plsc.subcore_barrier()

        def fire_gathers(b, j):
            pltpu.async_copy(q_hbm.at[idx.at[1, j]], qb.at[b], gsem[b])
            pltpu.async_copy(kv_hbm.at[idx.at[0, j]], kvb.at[b], gsem[b])

        for b in range(NBUF):
            fire_gathers(b, b)

        def grp(g, carry):
            for b in range(NBUF):
                j = g * NBUF + b
                pltpu.make_async_copy(q_hbm.at[idx.at[1, j]], qb.at[b],
                                      gsem[b]).wait()
                pltpu.make_async_copy(kv_hbm.at[idx.at[0, j]], kvb.at[b],
                                      gsem[b]).wait()

                @pl.when(g > 0)
                def _():
                    pltpu.make_async_copy(ob.at[b], acc.at[idx.at[1, j]],
                                          ssem[b]).wait()

                for r in range(CHUNK):
                    row = plsc.bitcast(kvb[b, r, :], jnp.bfloat16)
                    kvec, vvec = plsc.unpack(
                        row, format=plsc.PackFormat.INTERLEAVED)
                    prod = qb[b, r, :] * kvec
                    s = jnp.sum(prod)
                    wvec = jnp.exp(jnp.full((L,), s, jnp.float32) * 0.25)
                    ob[b, r, 0:C] = vvec * wvec
                    ob[b, r, C:2 * C] = wvec

                pltpu.async_copy(ob.at[b], acc.at[idx.at[1, j]], ssem[b],
                                 add=True)

                @pl.when(g < CPT // NBUF - 1)
                def _():
                    fire_gathers(b, j + NBUF)

            return carry

        lax.fori_loop(0, CPT // NBUF, grp, 0)
        for b in range(NBUF):
            pltpu.make_async_copy(ob.at[b], acc.at[idx.at[1, CPT - NBUF + b]],
                                  ssem[b]).wait()
        plsc.subcore_barrier()
        pltpu.sync_copy(acc.at[pl.ds(sid * RPT, RPT)],
                        out_hbm.at[cid, pl.ds(sid * RPT, RPT)])

    return kern(q, kv, edges)


def _proj1(x, Wq, bq, Wk, bk, Wv, bv, Ws, bs):
    def body(x_r, wq, bqr, wk, bkr, wv, bvr, ws, bsr, q_o, kv_o, skip_o):
        xv = x_r[...]
        q_o[0:N, :] = jnp.dot(xv, wq[...], preferred_element_type=jnp.float32) + bqr[...]
        q_o[N:NP, :] = jnp.zeros((NP - N, C), jnp.float32)
        kk = jnp.dot(xv, wk[...], preferred_element_type=jnp.float32) + bkr[...]
        vv = jnp.dot(xv, wv[...], preferred_element_type=jnp.float32) + bvr[...]
        kb = jax.lax.bitcast_convert_type(kk.astype(jnp.bfloat16), jnp.uint16).astype(jnp.uint32)
        vb = jax.lax.bitcast_convert_type(vv.astype(jnp.bfloat16), jnp.uint16).astype(jnp.uint32)
        kv_o[0:N, :] = kb | (vb << 16)
        kv_o[N:NP, :] = jnp.zeros((NP - N, C), jnp.uint32)
        skip_o[...] = jnp.dot(xv, ws[...], preferred_element_type=jnp.float32) + bsr[...]

    return pl.pallas_call(
        body,
        out_shape=(
            jax.ShapeDtypeStruct((NP, C), jnp.float32),
            jax.ShapeDtypeStruct((NP, C), jnp.uint32),
            jax.ShapeDtypeStruct((N, C), jnp.float32),
        ),
    )(x, Wq, bq.reshape(1, C), Wk, bk.reshape(1, C),
      Wv, bv.reshape(1, C), Ws, bs.reshape(1, C))


def _combine_proj2(accs, skip, Wq, bq, Wk, bk, Wv, bv, Ws, bs):
    def body(a_r, s_r, wq, bqr, wk, bkr, wv, bvr, ws, bsr, q_o, kv_o, skip_o):
        numer = a_r[0, 0:N, 0:C] + a_r[1, 0:N, 0:C]
        den = a_r[0, 0:N, C:C + 1] + a_r[1, 0:N, C:C + 1]
        h = jax.nn.relu(numer / jnp.maximum(den, 1e-30) + s_r[...])
        q_o[0:N, :] = jnp.dot(h, wq[...], preferred_element_type=jnp.float32) + bqr[...]
        q_o[N:NP, :] = jnp.zeros((NP - N, C), jnp.float32)
        kk = jnp.dot(h, wk[...], preferred_element_type=jnp.float32) + bkr[...]
        vv = jnp.dot(h, wv[...], preferred_element_type=jnp.float32) + bvr[...]
        kb = jax.lax.bitcast_convert_type(kk.astype(jnp.bfloat16), jnp.uint16).astype(jnp.uint32)
        vb = jax.lax.bitcast_convert_type(vv.astype(jnp.bfloat16), jnp.uint16).astype(jnp.uint32)
        kv_o[0:N, :] = kb | (vb << 16)
        kv_o[N:NP, :] = jnp.zeros((NP - N, C), jnp.uint32)
        skip_o[...] = jnp.dot(h, ws[...], preferred_element_type=jnp.float32) + bsr[...]

    return pl.pallas_call(
        body,
        out_shape=(
            jax.ShapeDtypeStruct((NP, C), jnp.float32),
            jax.ShapeDtypeStruct((NP, C), jnp.uint32),
            jax.ShapeDtypeStruct((N, C), jnp.float32),
        ),
    )(accs, skip, Wq, bq.reshape(1, C), Wk, bk.reshape(1, C),
      Wv, bv.reshape(1, C), Ws, bs.reshape(1, C))


def _combine_mlp(accs, skip, W1, b1, W2, b2, W3, b3):
    def elu(t):
        return jnp.where(t > 0, t, jnp.exp(jnp.minimum(t, 0.0)) - 1.0)

    def body(a_r, s_r, w1, b1r, w2, b2r, w3, b3r, o_r):
        numer = a_r[0, 0:N, 0:C] + a_r[1, 0:N, 0:C]
        den = a_r[0, 0:N, C:C + 1] + a_r[1, 0:N, C:C + 1]
        h = jax.nn.relu(numer / jnp.maximum(den, 1e-30) + s_r[...])
        h = elu(jnp.dot(h, w1[...], preferred_element_type=jnp.float32) + b1r[...])
        h = elu(jnp.dot(h, w2[...], preferred_element_type=jnp.float32) + b2r[...])
        h = elu(jnp.dot(h, w3[...], preferred_element_type=jnp.float32) + b3r[...])
        m = jnp.max(h, axis=1, keepdims=True)
        o_r[...] = h - m - jnp.log(jnp.sum(jnp.exp(h - m), axis=1, keepdims=True))

    return pl.pallas_call(
        body,
        out_shape=jax.ShapeDtypeStruct((N, 10), jnp.float32),
    )(accs, skip, W1, b1.reshape(1, -1), W2, b2.reshape(1, -1),
      W3, b3.reshape(1, -1))


def kernel(x, edge_index, Wq1, bq1, Wk1, bk1, Wv1, bv1, Ws1, bs1,
           Wq2, bq2, Wk2, bk2, Wv2, bv2, Ws2, bs2,
           W1, b1, W2, b2, W3, b3):
    ei = edge_index.astype(jnp.int32)
    pad = jnp.full((2, E2 - E), N, jnp.int32)
    edges = jnp.concatenate([ei, pad], axis=1).reshape(2, NCH, CHUNK)
    q1, kv1, skip1 = _proj1(x.astype(jnp.float32), Wq1, bq1, Wk1, bk1,
                            Wv1, bv1, Ws1, bs1)
    acc1 = _edge_pass(q1, kv1, edges)
    q2, kv2, skip2 = _combine_proj2(acc1, skip1, Wq2, bq2, Wk2, bk2,
                                    Wv2, bv2, Ws2, bs2)
    acc2 = _edge_pass(q2, kv2, edges)
    return _combine_mlp(acc2, skip2, W1, b1, W2, b2, W3, b3)


# speedup vs baseline: 1.0879x; 1.0087x over previous
import functools

import jax
import jax.numpy as jnp
from jax import lax
from jax.experimental import pallas as pl
from jax.experimental.pallas import tpu as pltpu
from jax.experimental.pallas import tpu_sc as plsc

N = 10000
E = 320000
D = 128
C = 16

L = 16
NC = 2
NS = 16
NW = NC * NS
CHUNK = 128
NCH = 2560
E2 = NCH * CHUNK
CPT = NCH // NW
NBUF = 4
NP = 10240
RPT = NP // NS


def _edge_pass(q, kv, edges):
    mesh = plsc.VectorSubcoreMesh(core_axis_name="c", subcore_axis_name="s")

    @functools.partial(
        pl.kernel,
        mesh=mesh,
        compiler_params=pltpu.CompilerParams(
            needs_layout_passes=False, use_tc_tiling_on_sc=False),
        out_type=jax.ShapeDtypeStruct((NC, NP, 2 * C), jnp.float32),
        scratch_types=[
            pltpu.VMEM((2, CPT, CHUNK), jnp.int32),
            pltpu.VMEM((NBUF, CHUNK, C), jnp.float32),
            pltpu.VMEM((NBUF, CHUNK, C), jnp.uint32),
            pltpu.VMEM((NBUF, CHUNK, 2 * C), jnp.float32),
            pltpu.VMEM((RPT, 2 * C), jnp.float32),
            pltpu.VMEM_SHARED((NP, 2 * C), jnp.float32),
        ] + [pltpu.SemaphoreType.DMA] * 17,
    )
    def kern(q_hbm, kv_hbm, e_hbm, out_hbm,
             idx, qb, kvb, ob, zbuf, acc, *sems):
        gsem = list(sems[:NBUF])
        ssem = list(sems[NBUF:2 * NBUF])
        isem = sems[2 * NBUF]
        cid = lax.axis_index("c")
        sid = lax.axis_index("s")
        wid = sid * NC + cid

        pltpu.async_copy(e_hbm.at[:, pl.ds(wid * CPT, CPT)], idx, isem)

        def zrow(r, carry):
            zbuf[r, 0:C] = jnp.zeros((L,), jnp.float32)
            zbuf[r, C:2 * C] = jnp.zeros((L,), jnp.float32)
            return carry

        lax.fori_loop(0, RPT, zrow, 0)
        pltpu.sync_copy(zbuf, acc.at[pl.ds(sid * RPT, RPT)])
        pltpu.make_async_copy(e_hbm.at[:, pl.ds(wid * CPT, CPT)], idx,
                              isem).wait()
        plsc.subcore_barrier()

        def fire_gathers(b, j):
            pltpu.async_copy(q_hbm.at[idx.at[1, j]], qb.at[b], gsem[b])
            pltpu.async_copy(kv_hbm.at[idx.at[0, j]], kvb.at[b], gsem[b])

        for b in range(NBUF):
            fire_gathers(b, b)

        def grp(g, carry):
            for b in range(NBUF):
                j = g * NBUF + b
                pltpu.make_async_copy(q_hbm.at[idx.at[1, j]], qb.at[b],
                                      gsem[b]).wait()
                pltpu.make_async_copy(kv_hbm.at[idx.at[0, j]], kvb.at[b],
                                      gsem[b]).wait()

                @pl.when(g > 0)
                def _():
                    pltpu.make_async_copy(ob.at[b], acc.at[idx.at[1, j]],
                                          ssem[b]).wait()

                for r in range(CHUNK):
                    row = plsc.bitcast(kvb[b, r, :], jnp.bfloat16)
                    kvec, vvec = plsc.unpack(
                        row, format=plsc.PackFormat.INTERLEAVED)
                    prod = qb[b, r, :] * kvec
                    s = jnp.sum(prod)
                    wvec = jnp.exp(jnp.full((L,), s, jnp.float32) * 0.25)
                    ob[b, r, 0:C] = vvec * wvec
                    ob[b, r, C:2 * C] = wvec

                pltpu.async_copy(ob.at[b], acc.at[idx.at[1, j]], ssem[b],
                                 add=True)

                @pl.when(g < CPT // NBUF - 1)
                def _():
                    fire_gathers(b, j + NBUF)

            return carry

        lax.fori_loop(0, CPT // NBUF, grp, 0)
        for b in range(NBUF):
            pltpu.make_async_copy(ob.at[b], acc.at[idx.at[1, CPT - NBUF + b]],
                                  ssem[b]).wait()
        plsc.subcore_barrier()
        pltpu.sync_copy(acc.at[pl.ds(sid * RPT, RPT)],
                        out_hbm.at[cid, pl.ds(sid * RPT, RPT)])

    return kern(q, kv, edges)


def _proj1(x, Wq, bq, Wk, bk, Wv, bv, Ws, bs):
    def body(x_r, wq, bqr, wk, bkr, wv, bvr, ws, bsr, q_o, kv_o, skip_o):
        xv = x_r[...]
        q_o[0:N, :] = jnp.dot(xv, wq[...], preferred_element_type=jnp.float32) + bqr[...]
        q_o[N:NP, :] = jnp.zeros((NP - N, C), jnp.float32)
        kk = jnp.dot(xv, wk[...], preferred_element_type=jnp.float32) + bkr[...]
        vv = jnp.dot(xv, wv[...], preferred_element_type=jnp.float32) + bvr[...]
        kb = jax.lax.bitcast_convert_type(kk.astype(jnp.bfloat16), jnp.uint16).astype(jnp.uint32)
        vb = jax.lax.bitcast_convert_type(vv.astype(jnp.bfloat16), jnp.uint16).astype(jnp.uint32)
        kv_o[0:N, :] = kb | (vb << 16)
        kv_o[N:NP, :] = jnp.zeros((NP - N, C), jnp.uint32)
        skip_o[...] = jnp.dot(xv, ws[...], preferred_element_type=jnp.float32) + bsr[...]

    return pl.pallas_call(
        body,
        out_shape=(
            jax.ShapeDtypeStruct((NP, C), jnp.float32),
            jax.ShapeDtypeStruct((NP, C), jnp.uint32),
            jax.ShapeDtypeStruct((N, C), jnp.float32),
        ),
    )(x, Wq, bq.reshape(1, C), Wk, bk.reshape(1, C),
      Wv, bv.reshape(1, C), Ws, bs.reshape(1, C))


def _combine_proj2(accs, skip, Wq, bq, Wk, bk, Wv, bv, Ws, bs):
    def body(a_r, s_r, wq, bqr, wk, bkr, wv, bvr, ws, bsr, q_o, kv_o, skip_o):
        numer = a_r[0, 0:N, 0:C] + a_r[1, 0:N, 0:C]
        den = a_r[0, 0:N, C:C + 1] + a_r[1, 0:N, C:C + 1]
        h = jax.nn.relu(numer / jnp.maximum(den, 1e-30) + s_r[...])
        q_o[0:N, :] = jnp.dot(h, wq[...], preferred_element_type=jnp.float32) + bqr[...]
        q_o[N:NP, :] = jnp.zeros((NP - N, C), jnp.float32)
        kk = jnp.dot(h, wk[...], preferred_element_type=jnp.float32) + bkr[...]
        vv = jnp.dot(h, wv[...], preferred_element_type=jnp.float32) + bvr[...]
        kb = jax.lax.bitcast_convert_type(kk.astype(jnp.bfloat16), jnp.uint16).astype(jnp.uint32)
        vb = jax.lax.bitcast_convert_type(vv.astype(jnp.bfloat16), jnp.uint16).astype(jnp.uint32)
        kv_o[0:N, :] = kb | (vb << 16)
        kv_o[N:NP, :] = jnp.zeros((NP - N, C), jnp.uint32)
        skip_o[...] = jnp.dot(h, ws[...], preferred_element_type=jnp.float32) + bsr[...]

    return pl.pallas_call(
        body,
        out_shape=(
            jax.ShapeDtypeStruct((NP, C), jnp.float32),
            jax.ShapeDtypeStruct((NP, C), jnp.uint32),
            jax.ShapeDtypeStruct((N, C), jnp.float32),
        ),
    )(accs, skip, Wq, bq.reshape(1, C), Wk, bk.reshape(1, C),
      Wv, bv.reshape(1, C), Ws, bs.reshape(1, C))


def _combine_mlp(accs, skip, W1, b1, W2, b2, W3, b3):
    def elu(t):
        return jnp.where(t > 0, t, jnp.exp(jnp.minimum(t, 0.0)) - 1.0)

    def body(a_r, s_r, w1, b1r, w2, b2r, w3, b3r, o_r):
        numer = a_r[0, 0:N, 0:C] + a_r[1, 0:N, 0:C]
        den = a_r[0, 0:N, C:C + 1] + a_r[1, 0:N, C:C + 1]
        h = jax.nn.relu(numer / jnp.maximum(den, 1e-30) + s_r[...])
        h = elu(jnp.dot(h, w1[...], preferred_element_type=jnp.float32) + b1r[...])
        h = elu(jnp.dot(h, w2[...], preferred_element_type=jnp.float32) + b2r[...])
        h = elu(jnp.dot(h, w3[...], preferred_element_type=jnp.float32) + b3r[...])
        m = jnp.max(h, axis=1, keepdims=True)
        o_r[...] = h - m - jnp.log(jnp.sum(jnp.exp(h - m), axis=1, keepdims=True))

    return pl.pallas_call(
        body,
        out_shape=jax.ShapeDtypeStruct((N, 10), jnp.float32),
    )(accs, skip, W1, b1.reshape(1, -1), W2, b2.reshape(1, -1),
      W3, b3.reshape(1, -1))


def kernel(x, edge_index, Wq1, bq1, Wk1, bk1, Wv1, bv1, Ws1, bs1,
           Wq2, bq2, Wk2, bk2, Wv2, bv2, Ws2, bs2,
           W1, b1, W2, b2, W3, b3):
    ei = edge_index.astype(jnp.int32)
    pad = jnp.full((2, E2 - E), N, jnp.int32)
    edges = jnp.concatenate([ei, pad], axis=1).reshape(2, NCH, CHUNK)
    q1, kv1, skip1 = _proj1(x.astype(jnp.float32), Wq1, bq1, Wk1, bk1,
                            Wv1, bv1, Ws1, bs1)
    acc1 = _edge_pass(q1, kv1, edges)
    q2, kv2, skip2 = _combine_proj2(acc1, skip1, Wq2, bq2, Wk2, bk2,
                                    Wv2, bv2, Ws2, bs2)
    acc2 = _edge_pass(q2, kv2, edges)
    return _combine_mlp(acc2, skip2, W1, b1, W2, b2, W3, b3)


# speedup vs baseline: 1.0954x; 1.0069x over previous
import functools

import jax
import jax.numpy as jnp
from jax import lax
from jax.experimental import pallas as pl
from jax.experimental.pallas import tpu as pltpu
from jax.experimental.pallas import tpu_sc as plsc

N = 10000
E = 320000
D = 128
C = 16

L = 16
NC = 2
NS = 16
NW = NC * NS
CHUNK = 128
NCH = 2560
E2 = NCH * CHUNK
CPT = NCH // NW
NBUF = 4
NP = 10240
RPT = NP // NS


def _edge_pass(q, kv, edges):
    mesh = plsc.VectorSubcoreMesh(core_axis_name="c", subcore_axis_name="s")

    @functools.partial(
        pl.kernel,
        mesh=mesh,
        compiler_params=pltpu.CompilerParams(
            needs_layout_passes=False, use_tc_tiling_on_sc=False),
        out_type=jax.ShapeDtypeStruct((NC, NP, 2 * C), jnp.float32),
        scratch_types=[
            pltpu.VMEM((2, CPT, CHUNK), jnp.int32),
            pltpu.VMEM((NBUF, CHUNK, C), jnp.float32),
            pltpu.VMEM((NBUF, CHUNK, C), jnp.uint32),
            pltpu.VMEM((NBUF, CHUNK, 2 * C), jnp.float32),
            pltpu.VMEM((RPT, 2 * C), jnp.float32),
            pltpu.VMEM_SHARED((NP, 2 * C), jnp.float32),
        ] + [pltpu.SemaphoreType.DMA] * 17,
    )
    def kern(q_hbm, kv_hbm, e_hbm, out_hbm,
             idx, qb, kvb, ob, zbuf, acc, *sems):
        gsem = list(sems[:NBUF])
        ssem = list(sems[NBUF:2 * NBUF])
        isem = sems[2 * NBUF]
        cid = lax.axis_index("c")
        sid = lax.axis_index("s")
        wid = sid * NC + cid

        pltpu.async_copy(e_hbm.at[:, pl.ds(wid * CPT, CPT)], idx, isem)

        def zrow(r, carry):
            zbuf[r, 0:C] = jnp.zeros((L,), jnp.float32)
            zbuf[r, C:2 * C] = jnp.zeros((L,), jnp.float32)
            return carry

        lax.fori_loop(0, RPT, zrow, 0)
        pltpu.sync_copy(zbuf, acc.at[pl.ds(sid * RPT, RPT)])
        pltpu.make_async_copy(e_hbm.at[:, pl.ds(wid * CPT, CPT)], idx,
                              isem).wait()
        plsc.subcore_barrier()

        def fire_gathers(b, j):
            pltpu.async_copy(q_hbm.at[idx.at[1, j]], qb.at[b], gsem[b])
            pltpu.async_copy(kv_hbm.at[idx.at[0, j]], kvb.at[b], gsem[b])

        for b in range(NBUF):
            fire_gathers(b, b)

        def grp(g, carry):
            for b in range(NBUF):
                j = g * NBUF + b
                pltpu.make_async_copy(q_hbm.at[idx.at[1, j]], qb.at[b],
                                      gsem[b]).wait()
                pltpu.make_async_copy(kv_hbm.at[idx.at[0, j]], kvb.at[b],
                                      gsem[b]).wait()

                @pl.when(g > 0)
                def _():
                    pltpu.make_async_copy(ob.at[b], acc.at[idx.at[1, j]],
                                          ssem[b]).wait()

                for r in range(CHUNK):
                    row = plsc.bitcast(kvb[b, r, :], jnp.bfloat16)
                    kvec, vvec = plsc.unpack(
                        row, format=plsc.PackFormat.INTERLEAVED)
                    prod = qb[b, r, :] * kvec
                    s = jnp.sum(prod)
                    wvec = jnp.exp(jnp.full((L,), s, jnp.float32) * 0.25)
                    ob[b, r, 0:C] = vvec * wvec
                    ob[b, r, C:2 * C] = wvec

                pltpu.async_copy(ob.at[b], acc.at[idx.at[1, j]], ssem[b],
                                 add=True)

                @pl.when(g < CPT // NBUF - 1)
                def _():
                    fire_gathers(b, j + NBUF)

            return carry

        lax.fori_loop(0, CPT // NBUF, grp, 0)
        for b in range(NBUF):
            pltpu.make_async_copy(ob.at[b], acc.at[idx.at[1, CPT - NBUF + b]],
                                  ssem[b]).wait()
        plsc.subcore_barrier()
        pltpu.sync_copy(acc.at[pl.ds(sid * RPT, RPT)],
                        out_hbm.at[cid, pl.ds(sid * RPT, RPT)])

    return kern(q, kv, edges)


BLK = 1280


def _proj1(x, Wq, bq, Wk, bk, Wv, bv, Ws, bs):
    def body(x_r, wq, bqr, wk, bkr, wv, bvr, ws, bsr, q_o, kv_o, skip_o):
        xv = x_r[...]
        q_o[...] = jnp.dot(xv, wq[...], preferred_element_type=jnp.float32) + bqr[...]
        kk = jnp.dot(xv, wk[...], preferred_element_type=jnp.float32) + bkr[...]
        vv = jnp.dot(xv, wv[...], preferred_element_type=jnp.float32) + bvr[...]
        kb = jax.lax.bitcast_convert_type(kk.astype(jnp.bfloat16), jnp.uint16).astype(jnp.uint32)
        vb = jax.lax.bitcast_convert_type(vv.astype(jnp.bfloat16), jnp.uint16).astype(jnp.uint32)
        kv_o[...] = kb | (vb << 16)
        skip_o[...] = jnp.dot(xv, ws[...], preferred_element_type=jnp.float32) + bsr[...]

    w_spec = lambda shp: pl.BlockSpec(shp, lambda i: (0, 0))
    return pl.pallas_call(
        body,
        grid=(NP // BLK,),
        in_specs=[
            pl.BlockSpec((BLK, D), lambda i: (i, 0)),
            w_spec((D, C)), w_spec((1, C)), w_spec((D, C)), w_spec((1, C)),
            w_spec((D, C)), w_spec((1, C)), w_spec((D, C)), w_spec((1, C)),
        ],
        out_specs=(
            pl.BlockSpec((BLK, C), lambda i: (i, 0)),
            pl.BlockSpec((BLK, C), lambda i: (i, 0)),
            pl.BlockSpec((BLK, C), lambda i: (i, 0)),
        ),
        out_shape=(
            jax.ShapeDtypeStruct((NP, C), jnp.float32),
            jax.ShapeDtypeStruct((NP, C), jnp.uint32),
            jax.ShapeDtypeStruct((N, C), jnp.float32),
        ),
    )(x, Wq, bq.reshape(1, C), Wk, bk.reshape(1, C),
      Wv, bv.reshape(1, C), Ws, bs.reshape(1, C))


def _combine_proj2(accs, skip, Wq, bq, Wk, bk, Wv, bv, Ws, bs):
    def body(a_r, s_r, wq, bqr, wk, bkr, wv, bvr, ws, bsr, q_o, kv_o, skip_o):
        numer = a_r[0, :, 0:C] + a_r[1, :, 0:C]
        den = a_r[0, :, C:C + 1] + a_r[1, :, C:C + 1]
        h = jax.nn.relu(numer / jnp.maximum(den, 1e-30) + s_r[...])
        q_o[...] = jnp.dot(h, wq[...], preferred_element_type=jnp.float32) + bqr[...]
        kk = jnp.dot(h, wk[...], preferred_element_type=jnp.float32) + bkr[...]
        vv = jnp.dot(h, wv[...], preferred_element_type=jnp.float32) + bvr[...]
        kb = jax.lax.bitcast_convert_type(kk.astype(jnp.bfloat16), jnp.uint16).astype(jnp.uint32)
        vb = jax.lax.bitcast_convert_type(vv.astype(jnp.bfloat16), jnp.uint16).astype(jnp.uint32)
        kv_o[...] = kb | (vb << 16)
        skip_o[...] = jnp.dot(h, ws[...], preferred_element_type=jnp.float32) + bsr[...]

    w_spec = lambda shp: pl.BlockSpec(shp, lambda i: (0, 0))
    return pl.pallas_call(
        body,
        grid=(NP // BLK,),
        in_specs=[
            pl.BlockSpec((2, BLK, 2 * C), lambda i: (0, i, 0)),
            pl.BlockSpec((BLK, C), lambda i: (i, 0)),
            w_spec((C, C)), w_spec((1, C)), w_spec((C, C)), w_spec((1, C)),
            w_spec((C, C)), w_spec((1, C)), w_spec((C, C)), w_spec((1, C)),
        ],
        out_specs=(
            pl.BlockSpec((BLK, C), lambda i: (i, 0)),
            pl.BlockSpec((BLK, C), lambda i: (i, 0)),
            pl.BlockSpec((BLK, C), lambda i: (i, 0)),
        ),
        out_shape=(
            jax.ShapeDtypeStruct((NP, C), jnp.float32),
            jax.ShapeDtypeStruct((NP, C), jnp.uint32),
            jax.ShapeDtypeStruct((N, C), jnp.float32),
        ),
    )(accs, skip, Wq, bq.reshape(1, C), Wk, bk.reshape(1, C),
      Wv, bv.reshape(1, C), Ws, bs.reshape(1, C))


def _combine_mlp(accs, skip, W1, b1, W2, b2, W3, b3):
    def elu(t):
        return jnp.where(t > 0, t, jnp.exp(jnp.minimum(t, 0.0)) - 1.0)

    def body(a_r, s_r, w1, b1r, w2, b2r, w3, b3r, o_r):
        numer = a_r[0, :, 0:C] + a_r[1, :, 0:C]
        den = a_r[0, :, C:C + 1] + a_r[1, :, C:C + 1]
        h = jax.nn.relu(numer / jnp.maximum(den, 1e-30) + s_r[...])
        h = elu(jnp.dot(h, w1[...], preferred_element_type=jnp.float32) + b1r[...])
        h = elu(jnp.dot(h, w2[...], preferred_element_type=jnp.float32) + b2r[...])
        h = elu(jnp.dot(h, w3[...], preferred_element_type=jnp.float32) + b3r[...])
        m = jnp.max(h, axis=1, keepdims=True)
        o_r[...] = h - m - jnp.log(jnp.sum(jnp.exp(h - m), axis=1, keepdims=True))

    w_spec = lambda shp: pl.BlockSpec(shp, lambda i: (0, 0))
    return pl.pallas_call(
        body,
        grid=(NP // BLK,),
        in_specs=[
            pl.BlockSpec((2, BLK, 2 * C), lambda i: (0, i, 0)),
            pl.BlockSpec((BLK, C), lambda i: (i, 0)),
            w_spec((C, 64)), w_spec((1, 64)), w_spec((64, 128)), w_spec((1, 128)),
            w_spec((128, 10)), w_spec((1, 10)),
        ],
        out_specs=pl.BlockSpec((BLK, 10), lambda i: (i, 0)),
        out_shape=jax.ShapeDtypeStruct((N, 10), jnp.float32),
    )(accs, skip, W1, b1.reshape(1, -1), W2, b2.reshape(1, -1),
      W3, b3.reshape(1, -1))


def kernel(x, edge_index, Wq1, bq1, Wk1, bk1, Wv1, bv1, Ws1, bs1,
           Wq2, bq2, Wk2, bk2, Wv2, bv2, Ws2, bs2,
           W1, b1, W2, b2, W3, b3):
    ei = edge_index.astype(jnp.int32)
    pad = jnp.full((2, E2 - E), N, jnp.int32)
    edges = jnp.concatenate([ei, pad], axis=1).reshape(2, NCH, CHUNK)
    q1, kv1, skip1 = _proj1(x.astype(jnp.float32), Wq1, bq1, Wk1, bk1,
                            Wv1, bv1, Ws1, bs1)
    acc1 = _edge_pass(q1, kv1, edges)
    q2, kv2, skip2 = _combine_proj2(acc1, skip1, Wq2, bq2, Wk2, bk2,
                                    Wv2, bv2, Ws2, bs2)
    acc2 = _edge_pass(q2, kv2, edges)
    return _combine_mlp(acc2, skip2, W1, b1, W2, b2, W3, b3)
